# baseline (speedup 1.0000x reference)
import functools

import jax
import jax.numpy as jnp
from jax import lax
from jax.experimental import pallas as pl
from jax.experimental.pallas import tpu as pltpu
from jax.experimental.pallas import tpu_sc as plsc


@functools.lru_cache(maxsize=None)
def _build(B, T, V):
    info = plsc.get_sparse_core_info()
    NC, NS, L = info.num_cores, info.num_subcores, info.num_lanes
    NW = NC * NS
    RPW = 2 * L

    assert B == NW * RPW, B
    n_pw = RPW * T
    n_vchunks = n_pw // L

    mesh = plsc.VectorSubcoreMesh(core_axis_name="c", subcore_axis_name="s")

    @functools.partial(
        pl.kernel,
        mesh=mesh,
        out_type=jax.ShapeDtypeStruct((NW, 2 * L), jnp.float32),
        scratch_types=[
            pltpu.VMEM((n_pw,), jnp.int32),
            pltpu.VMEM((n_pw,), jnp.int32),
            pltpu.VMEM((n_pw,), jnp.int32),
            pltpu.VMEM((n_pw,), jnp.float32),
            pltpu.VMEM((2 * L,), jnp.float32),
            pltpu.SemaphoreType.DMA,
        ],
    )
    def sc_loss(in_hbm, tgt_hbm, out_hbm, pos_v, tgt_v, idx_v, val_v,
                res_v, sem):
        w = lax.axis_index("s") * NC + lax.axis_index("c")
        row0 = w * RPW
        it = lax.iota(jnp.int32, L)

        def build_pos(j, carry):
            t = j // 2
            r = (j % 2) * L + it
            pos_v[pl.ds(j * L, L)] = (row0 + r) * T + t
            return carry

        lax.fori_loop(0, n_vchunks, build_pos, 0)

        pltpu.async_copy(tgt_hbm.at[pos_v], tgt_v, sem).wait()

        def build_idx(j, carry):
            sl = pl.ds(j * L, L)
            idx_v[sl] = pos_v[sl] * V + tgt_v[sl]
            return carry

        lax.fori_loop(0, n_vchunks, build_idx, 0)

        value_gather = pltpu.async_copy(in_hbm.at[idx_v], val_v, sem)

        def count_step(t, nnz):
            nnz0, nnz1 = nnz
            one = jnp.ones((L,), jnp.int32)
            zero = jnp.zeros((L,), jnp.int32)
            nnz0 = nnz0 + jnp.where(tgt_v[pl.ds(t * 2 * L, L)] > 0, one, zero)
            nnz1 = nnz1 + jnp.where(tgt_v[pl.ds(t * 2 * L + L, L)] > 0,
                                    one, zero)
            return nnz0, nnz1

        zi = jnp.zeros((L,), jnp.int32)
        nnz0, nnz1 = lax.fori_loop(0, T, count_step, (zi, zi))
        lim0 = jnp.minimum(nnz0 + 1, T)
        lim1 = jnp.minimum(nnz1 + 1, T)

        value_gather.wait()

        def sum_step(t, acc):
            acc0, acc1 = acc
            zf = jnp.zeros((L,), jnp.float32)
            v0 = val_v[pl.ds(t * 2 * L, L)]
            v1 = val_v[pl.ds(t * 2 * L + L, L)]
            acc0 = acc0 - jnp.where(t < lim0, v0, zf)
            acc1 = acc1 - jnp.where(t < lim1, v1, zf)
            return acc0, acc1

        zf = jnp.zeros((L,), jnp.float32)
        acc0, acc1 = lax.fori_loop(0, T, sum_step, (zf, zf))

        res_v[pl.ds(0, L)] = acc0 + acc1
        res_v[pl.ds(L, L)] = (lim0 + lim1).astype(jnp.float32)
        pltpu.sync_copy(res_v, out_hbm.at[w])

    return sc_loss, L


def kernel(input, target):
    B, T, V = input.shape
    sc_loss, L = _build(B, T, V)
    out = sc_loss(input.reshape(-1), target.reshape(-1).astype(jnp.int32))
    return jnp.sum(out[:, :L]) / jnp.sum(out[:, L:])


# speedup vs baseline: 1.5410x; 1.5410x over previous
import functools

import jax
import jax.numpy as jnp
from jax import lax
from jax.experimental import pallas as pl
from jax.experimental.pallas import tpu as pltpu
from jax.experimental.pallas import tpu_sc as plsc


@functools.lru_cache(maxsize=None)
def _build(B, T, V):
    info = plsc.get_sparse_core_info()
    NC, NS, L = info.num_cores, info.num_subcores, info.num_lanes
    NW = NC * NS
    RPW = B // NW

    assert RPW % 2 == 0 and (RPW * T) % L == 0
    n_pw = RPW * T
    t_chunks = -(-T // L)

    mesh = plsc.VectorSubcoreMesh(core_axis_name="c", subcore_axis_name="s")

    @functools.partial(
        pl.kernel,
        mesh=mesh,
        out_type=jax.ShapeDtypeStruct((NW * 2 * L,), jnp.float32),
        scratch_types=[
            pltpu.VMEM((n_pw,), jnp.int32),
            pltpu.VMEM((RPW * L,), jnp.int32),
            pltpu.VMEM((T, V), jnp.float32),
            pltpu.VMEM((T, V), jnp.float32),
            pltpu.VMEM((2 * L,), jnp.float32),
            pltpu.SemaphoreType.DMA,
            pltpu.SemaphoreType.DMA,
        ],
        compiler_params=pltpu.CompilerParams(
            use_tc_tiling_on_sc=True,
            needs_layout_passes=False,
        ),
    )
    def sc_loss(in_hbm, tgt_hbm, out_hbm, tgt_v, lim_v, slab0, slab1,
                res_v, sem0, sem1):
        w = lax.axis_index("s") * NC + lax.axis_index("c")
        b0 = w * RPW
        it = lax.iota(jnp.int32, L)

        slabs = (slab0, slab1)
        sems = (sem0, sem1)

        for d in range(2):
            pltpu.async_copy(in_hbm.at[b0 + d], slabs[d], sems[d])

        pltpu.sync_copy(tgt_hbm.at[pl.ds(w * n_pw, n_pw)], tgt_v)

        def row_count(r, carry):
            nnz = jnp.zeros((L,), jnp.int32)
            for c in range(t_chunks):
                pos = c * L + it
                valid = pos < T
                tv = plsc.load_gather(tgt_v, [r * T + pos], mask=valid)
                nnz = nnz + plsc.all_reduce_population_count(
                    valid & (tv > 0))
            lim_v[pl.ds(r * L, L)] = jnp.minimum(nnz + 1, T)
            return carry

        lax.fori_loop(0, RPW, row_count, 0)

        def consume(r, sv, acc):
            lim = lim_v[pl.ds(r * L, L)]
            for c in range(t_chunks):
                pos = c * L + it
                valid = pos < T
                tgt16 = plsc.load_gather(tgt_v, [r * T + pos], mask=valid)
                m = pos < lim
                vals = plsc.load_gather(sv, [pos, tgt16], mask=m)
                acc = acc - jnp.where(m, vals, jnp.zeros((L,), jnp.float32))
            return acc

        def pair_step(k, acc):
            for d in range(2):
                r = 2 * k + d
                pltpu.make_async_copy(in_hbm.at[b0 + r], slabs[d],
                                      sems[d]).wait()
                acc = consume(r, slabs[d], acc)

                @pl.when(r + 2 < RPW)
                def _():
                    pltpu.async_copy(in_hbm.at[b0 + r + 2], slabs[d],
                                     sems[d])
            return acc

        acc = lax.fori_loop(0, RPW // 2, pair_step,
                            jnp.zeros((L,), jnp.float32))

        def mask_total(r, macc):
            return macc + lim_v[pl.ds(r * L, L)].astype(jnp.float32)

        macc = lax.fori_loop(0, RPW, mask_total,
                             jnp.zeros((L,), jnp.float32))

        res_v[pl.ds(0, L)] = acc
        res_v[pl.ds(L, L)] = macc
        pltpu.sync_copy(res_v, out_hbm.at[pl.ds(w * 2 * L, 2 * L)])

    return sc_loss, L


def kernel(input, target):
    B, T, V = input.shape
    sc_loss, L = _build(B, T, V)
    out = sc_loss(input, target.reshape(-1).astype(jnp.int32))
    out = out.reshape(-1, 2 * L)
    return jnp.sum(out[:, :L]) / (jnp.sum(out[:, L:]) / L)
